# Initial kernel scaffold; baseline (speedup 1.0000x reference)
import functools

import jax
import jax.numpy as jnp
from jax import lax
from jax.experimental import pallas as pl
from jax.experimental.pallas import tpu as pltpu
from jax.experimental.pallas import tpu_sc as plsc

B = 4096
D = 2048
NUM_LEAVES = 1000
NUM_SUPER = 100
NUM_NODES = 1 + NUM_SUPER + NUM_LEAVES
N_PAD = 1152
TEMPERATURE = 1.0

SC_CORES = 2
SC_SUBCORES = 16
SC_WORKERS = SC_CORES * SC_SUBCORES
ROWS_PER_WORKER = B // SC_WORKERS
SC_CHUNK = 32
SC_NCHUNK = ROWS_PER_WORKER // SC_CHUNK
OUT_CHUNK = 8
N_OUT_CHUNKS = NUM_LEAVES // OUT_CHUNK
OUT_CHUNKS_PER_TILE = -(-N_OUT_CHUNKS // SC_SUBCORES)

BM = 512


def _sc_scatter_body(feat_hbm, lab_hbm, zeros_hbm, out_hbm,
                     idx_v, rows_v, zrows_v, acc_shared):
    c = lax.axis_index("c")
    s = lax.axis_index("s")
    base = (c * SC_SUBCORES + s) * ROWS_PER_WORKER
    pltpu.sync_copy(zeros_hbm, zrows_v)
    for j in range(OUT_CHUNKS_PER_TILE):
        cid = s * OUT_CHUNKS_PER_TILE + j

        @pl.when(cid < N_OUT_CHUNKS)
        def _():
            pltpu.sync_copy(zrows_v, acc_shared.at[pl.ds(cid * OUT_CHUNK, OUT_CHUNK)])

    plsc.subcore_barrier()
    for j in range(SC_NCHUNK):
        off = base + j * SC_CHUNK
        pltpu.sync_copy(lab_hbm.at[pl.ds(off, SC_CHUNK)], idx_v)
        pltpu.sync_copy(feat_hbm.at[pl.ds(off, SC_CHUNK)], rows_v)
        pltpu.sync_copy(rows_v, acc_shared.at[idx_v], add=True)
    plsc.subcore_barrier()
    for j in range(OUT_CHUNKS_PER_TILE):
        cid = s * OUT_CHUNKS_PER_TILE + j

        @pl.when(cid < N_OUT_CHUNKS)
        def _():
            pltpu.sync_copy(
                acc_shared.at[pl.ds(cid * OUT_CHUNK, OUT_CHUNK)],
                out_hbm.at[pl.ds(c * NUM_LEAVES + cid * OUT_CHUNK, OUT_CHUNK)])


_sc_scatter = pl.kernel(
    _sc_scatter_body,
    out_type=jax.ShapeDtypeStruct((SC_CORES * NUM_LEAVES, D), jnp.float32),
    mesh=plsc.VectorSubcoreMesh(core_axis_name="c", subcore_axis_name="s"),
    scratch_types=[
        pltpu.VMEM((SC_CHUNK,), jnp.int32),
        pltpu.VMEM((SC_CHUNK, D), jnp.float32),
        pltpu.VMEM((OUT_CHUNK, D), jnp.float32),
        pltpu.VMEM_SHARED((NUM_LEAVES, D), jnp.float32),
    ],
)


def _vmf_weights(ave, amount, sums, counts):
    new_amount = amount + counts
    new_ave = (ave * amount + sums) / new_amount
    r2 = jnp.sum(new_ave * new_ave, axis=1, keepdims=True)
    r = jnp.sqrt(r2)
    r_c = jnp.clip(r, 1e-6, 1.0 - 1e-6)
    kappa = r_c * (D - r_c * r_c) / (1.0 - r_c * r_c)
    scale = kappa / jnp.maximum(r, 1e-12) / TEMPERATURE
    w = new_ave * scale
    pt = jnp.sum(jnp.sum(sums * new_ave, axis=1, keepdims=True) * scale)
    return w, pt


def _prep_body(lab_ref, p_ref, ave_root_ref, ave_super_ref, ave_leaf_ref,
               amt_root_ref, amt_super_ref, amt_leaf_ref, w_ref, pt_ref):
    lab = lab_ref[...]
    node_iota = lax.broadcasted_iota(jnp.int32, (1024, B), 0)
    onehot = (node_iota == lab).astype(jnp.float32)
    counts_all = jnp.sum(onehot, axis=1, keepdims=True)
    counts_leaf = counts_all[:NUM_LEAVES]

    leaf_sums = p_ref[0] + p_ref[1]

    io_s = lax.broadcasted_iota(jnp.int32, (NUM_SUPER, NUM_LEAVES), 0)
    io_l = lax.broadcasted_iota(jnp.int32, (NUM_SUPER, NUM_LEAVES), 1)
    sel = ((io_l >= 10 * io_s) & (io_l < 10 * io_s + 10)).astype(jnp.float32)
    super_sums = jnp.dot(sel, leaf_sums, preferred_element_type=jnp.float32)
    super_counts = jnp.dot(sel, counts_leaf, preferred_element_type=jnp.float32)
    root_sum = jnp.sum(super_sums, axis=0, keepdims=True)
    root_count = jnp.sum(counts_leaf, axis=0, keepdims=True)

    w_root, pt0 = _vmf_weights(ave_root_ref[...], amt_root_ref[...], root_sum, root_count)
    w_super, pt1 = _vmf_weights(ave_super_ref[...], amt_super_ref[...], super_sums, super_counts)
    w_leaf, pt2 = _vmf_weights(ave_leaf_ref[...], amt_leaf_ref[...], leaf_sums, counts_leaf)

    w_ref[...] = jnp.concatenate(
        [w_root, w_super, w_leaf,
         jnp.zeros((N_PAD - NUM_NODES, D), jnp.float32)], axis=0)
    pt_ref[0, 0] = pt0 + pt1 + pt2


def _loss_body(feat_ref, w_ref, pt_ref, out_ref, acc_ref):
    i = pl.program_id(0)

    @pl.when(i == 0)
    def _():
        acc_ref[0, 0] = 0.0

    z = lax.dot_general(
        feat_ref[...], w_ref[...],
        dimension_numbers=(((1,), (1,)), ((), ())),
        preferred_element_type=jnp.float32)
    sp = jnp.maximum(z, 0.0) + jnp.log1p(jnp.exp(-jnp.abs(z)))
    mask = lax.broadcasted_iota(jnp.int32, (BM, N_PAD), 1) < NUM_NODES
    acc_ref[0, 0] += jnp.sum(jnp.where(mask, sp, 0.0))

    @pl.when(i == pl.num_programs(0) - 1)
    def _():
        out_ref[0, 0] = (acc_ref[0, 0] - pt_ref[0, 0]) / float(B * NUM_NODES)


def _run_prep(labels, partials, ave, amount):
    lab2 = labels.reshape(1, B).astype(jnp.int32)
    amount_col = amount.reshape(NUM_NODES, 1)
    return pl.pallas_call(
        _prep_body,
        out_shape=(
            jax.ShapeDtypeStruct((N_PAD, D), jnp.float32),
            jax.ShapeDtypeStruct((1, 1), jnp.float32),
        ),
    )(lab2, partials.reshape(SC_CORES, NUM_LEAVES, D),
      ave[0:1], ave[1:1 + NUM_SUPER], ave[1 + NUM_SUPER:NUM_NODES],
      amount_col[0:1], amount_col[1:1 + NUM_SUPER],
      amount_col[1 + NUM_SUPER:NUM_NODES])


def _run_loss(features, w, pt):
    grid = (B // BM,)
    return pl.pallas_call(
        _loss_body,
        grid=grid,
        in_specs=[
            pl.BlockSpec((BM, D), lambda i: (i, 0)),
            pl.BlockSpec((N_PAD, D), lambda i: (0, 0)),
            pl.BlockSpec((1, 1), lambda i: (0, 0)),
        ],
        out_specs=pl.BlockSpec((1, 1), lambda i: (0, 0)),
        out_shape=jax.ShapeDtypeStruct((1, 1), jnp.float32),
        scratch_shapes=[pltpu.SMEM((1, 1), jnp.float32)],
    )(features, w, pt)


def kernel(features, leaf_labels, Ave, Amount):
    zeros_block = jnp.zeros((OUT_CHUNK, D), jnp.float32)
    partials = _sc_scatter(features, leaf_labels.astype(jnp.int32), zeros_block)
    w, pt = _run_prep(leaf_labels, partials, Ave, Amount)
    out = _run_loss(features, w, pt)
    return out.reshape(())


# speedup vs baseline: 7.8915x; 7.8915x over previous
import functools

import jax
import jax.numpy as jnp
from jax import lax
from jax.experimental import pallas as pl
from jax.experimental.pallas import tpu as pltpu
from jax.experimental.pallas import tpu_sc as plsc

B = 4096
D = 2048
NUM_LEAVES = 1000
NUM_SUPER = 100
NUM_NODES = 1 + NUM_SUPER + NUM_LEAVES
N_PAD = 1152
TEMPERATURE = 1.0

SC_CORES = 2
SC_SUBCORES = 16
WIN = D // SC_SUBCORES
NODE_SPLIT = 512
SC_CHUNK = 128
SC_NCHUNK = B // SC_CHUNK

BM = 512


def _sc_scatter_body(feat_hbm, lab_hbm, zeros_hbm, out_hbm, idx_v, rows_v, acc):
    c = lax.axis_index("c")
    s = lax.axis_index("s")
    col = s * WIN
    nbase = c * NODE_SPLIT
    bound = jnp.where(c == 0, NODE_SPLIT, NUM_LEAVES - NODE_SPLIT)
    pltpu.sync_copy(zeros_hbm, acc)
    iota16 = lax.iota(jnp.int32, 16)
    z16 = jnp.zeros((16,), jnp.int32)

    def chunk_body(j, carry):
        pltpu.sync_copy(lab_hbm.at[pl.ds(j, 1)], idx_v)
        pltpu.sync_copy(
            feat_hbm.at[pl.ds(j * SC_CHUNK, SC_CHUNK), pl.ds(col, WIN)], rows_v)
        for r in range(SC_CHUNK):
            labr = plsc.load_gather(
                idx_v, [z16, jnp.full((16,), r, jnp.int32)]) - nbase
            m = (labr >= 0) & (labr < bound)
            for k in range(WIN // 16):
                v = rows_v[r, pl.ds(k * 16, 16)]
                plsc.addupdate_scatter(acc, [labr, iota16 + (k * 16)], v, mask=m)
        return carry

    lax.fori_loop(0, SC_NCHUNK, chunk_body, 0)
    pltpu.sync_copy(
        acc.at[pl.ds(0, 488)],
        out_hbm.at[pl.ds(c * NODE_SPLIT, 488), pl.ds(col, WIN)])

    @pl.when(c == 0)
    def _():
        pltpu.sync_copy(
            acc.at[pl.ds(488, 24)],
            out_hbm.at[pl.ds(488, 24), pl.ds(col, WIN)])


@functools.lru_cache(maxsize=1)
def _sc_scatter():
    return pl.kernel(
        _sc_scatter_body,
        out_type=jax.ShapeDtypeStruct((NUM_LEAVES, D), jnp.float32),
        mesh=plsc.VectorSubcoreMesh(core_axis_name="c", subcore_axis_name="s"),
        compiler_params=pltpu.CompilerParams(needs_layout_passes=False),
        scratch_types=[
            pltpu.VMEM((1, SC_CHUNK), jnp.int32),
            pltpu.VMEM((SC_CHUNK, WIN), jnp.float32),
            pltpu.VMEM((NODE_SPLIT, WIN), jnp.float32),
        ],
    )


def _vmf_weights(ave, amount, sums, counts):
    new_amount = amount + counts
    new_ave = (ave * amount + sums) / new_amount
    r2 = jnp.sum(new_ave * new_ave, axis=1, keepdims=True)
    r = jnp.sqrt(r2)
    r_c = jnp.clip(r, 1e-6, 1.0 - 1e-6)
    kappa = r_c * (D - r_c * r_c) / (1.0 - r_c * r_c)
    scale = kappa / jnp.maximum(r, 1e-12) / TEMPERATURE
    w = new_ave * scale
    pt = jnp.sum(jnp.sum(sums * new_ave, axis=1, keepdims=True) * scale)
    return w, pt


def _prep_body(lab_ref, p_ref, ave_root_ref, ave_super_ref, ave_leaf_ref,
               amt_root_ref, amt_super_ref, amt_leaf_ref, w_ref, pt_ref):
    lab = lab_ref[...]
    node_iota = lax.broadcasted_iota(jnp.int32, (1024, B), 0)
    onehot = (node_iota == lab).astype(jnp.float32)
    counts_all = jnp.sum(onehot, axis=1, keepdims=True)
    counts_leaf = counts_all[:NUM_LEAVES]

    leaf_sums = p_ref[...]

    io_s = lax.broadcasted_iota(jnp.int32, (NUM_SUPER, NUM_LEAVES), 0)
    io_l = lax.broadcasted_iota(jnp.int32, (NUM_SUPER, NUM_LEAVES), 1)
    sel = ((io_l >= 10 * io_s) & (io_l < 10 * io_s + 10)).astype(jnp.float32)
    super_sums = jnp.dot(sel, leaf_sums, preferred_element_type=jnp.float32)
    super_counts = jnp.dot(sel, counts_leaf, preferred_element_type=jnp.float32)
    root_sum = jnp.sum(super_sums, axis=0, keepdims=True)
    root_count = jnp.sum(counts_leaf, axis=0, keepdims=True)

    w_root, pt0 = _vmf_weights(ave_root_ref[...], amt_root_ref[...], root_sum, root_count)
    w_super, pt1 = _vmf_weights(ave_super_ref[...], amt_super_ref[...], super_sums, super_counts)
    w_leaf, pt2 = _vmf_weights(ave_leaf_ref[...], amt_leaf_ref[...], leaf_sums, counts_leaf)

    w_ref[...] = jnp.concatenate(
        [w_root, w_super, w_leaf,
         jnp.zeros((N_PAD - NUM_NODES, D), jnp.float32)], axis=0)
    pt_ref[0, 0] = pt0 + pt1 + pt2


def _loss_body(feat_ref, w_ref, pt_ref, out_ref, acc_ref):
    i = pl.program_id(0)

    @pl.when(i == 0)
    def _():
        acc_ref[0, 0] = 0.0

    z = lax.dot_general(
        feat_ref[...], w_ref[...],
        dimension_numbers=(((1,), (1,)), ((), ())),
        preferred_element_type=jnp.float32)
    sp = jnp.maximum(z, 0.0) + jnp.log1p(jnp.exp(-jnp.abs(z)))
    mask = lax.broadcasted_iota(jnp.int32, (BM, N_PAD), 1) < NUM_NODES
    acc_ref[0, 0] += jnp.sum(jnp.where(mask, sp, 0.0))

    @pl.when(i == pl.num_programs(0) - 1)
    def _():
        val = (acc_ref[0, 0] - pt_ref[0, 0]) / float(B * NUM_NODES)
        out_ref[...] = jnp.full((1, 1), val, jnp.float32)


def _run_prep(labels, partials, ave, amount):
    lab2 = labels.reshape(1, B).astype(jnp.int32)
    amount_col = amount.reshape(NUM_NODES, 1)
    return pl.pallas_call(
        _prep_body,
        out_shape=(
            jax.ShapeDtypeStruct((N_PAD, D), jnp.float32),
            jax.ShapeDtypeStruct((1, 1), jnp.float32),
        ),
        out_specs=(
            pl.BlockSpec(memory_space=pltpu.VMEM),
            pl.BlockSpec(memory_space=pltpu.SMEM),
        ),
    )(lab2, partials,
      ave[0:1], ave[1:1 + NUM_SUPER], ave[1 + NUM_SUPER:NUM_NODES],
      amount_col[0:1], amount_col[1:1 + NUM_SUPER],
      amount_col[1 + NUM_SUPER:NUM_NODES])


def _run_loss(features, w, pt):
    grid = (B // BM,)
    return pl.pallas_call(
        _loss_body,
        grid=grid,
        in_specs=[
            pl.BlockSpec((BM, D), lambda i: (i, 0)),
            pl.BlockSpec((N_PAD, D), lambda i: (0, 0)),
            pl.BlockSpec(memory_space=pltpu.SMEM),
        ],
        out_specs=pl.BlockSpec((1, 1), lambda i: (0, 0)),
        out_shape=jax.ShapeDtypeStruct((1, 1), jnp.float32),
        scratch_shapes=[pltpu.SMEM((1, 1), jnp.float32)],
    )(features, w, pt)


def kernel(features, leaf_labels, Ave, Amount):
    zeros_block = jnp.zeros((NODE_SPLIT, WIN), jnp.float32)
    labs2 = leaf_labels.reshape(SC_NCHUNK, SC_CHUNK).astype(jnp.int32)
    leaf_sums = _sc_scatter()(features, labs2, zeros_block)
    w, pt = _run_prep(leaf_labels, leaf_sums, Ave, Amount)
    out = _run_loss(features, w, pt)
    return out.reshape(())


# speedup vs baseline: 10.9118x; 1.3827x over previous
import functools

import jax
import jax.numpy as jnp
from jax import lax
from jax.experimental import pallas as pl
from jax.experimental.pallas import tpu as pltpu
from jax.experimental.pallas import tpu_sc as plsc

B = 4096
D = 2048
NUM_LEAVES = 1000
NUM_SUPER = 100
NUM_NODES = 1 + NUM_SUPER + NUM_LEAVES
N_PAD = 1152
TEMPERATURE = 1.0

SC_CORES = 2
SC_SUBCORES = 16
WIN = D // SC_SUBCORES
ROWS_PER_CORE = B // SC_CORES
SC_CHUNK = 8
LROWS = ROWS_PER_CORE // 128

BM = 512


def _sc_scatter_body(feat_hbm, lab_hbm, zeros_hbm, out_hbm,
                     idx_v, rows_a, rows_b, sem_a, sem_b, acc):
    c = lax.axis_index("c")
    s = lax.axis_index("s")
    col = s * WIN
    pltpu.sync_copy(zeros_hbm, acc)
    iota16 = lax.iota(jnp.int32, 16)
    z16 = jnp.zeros((16,), jnp.int32)
    bufs = (rows_a, rows_b)
    sems = (sem_a, sem_b)

    def lr_body(lr, carry):
        pltpu.sync_copy(lab_hbm.at[pl.ds(c * LROWS + lr, 1)], idx_v)
        row0 = c * ROWS_PER_CORE + lr * 128

        def start(q):
            return pltpu.async_copy(
                feat_hbm.at[pl.ds(row0 + q * SC_CHUNK, SC_CHUNK),
                            pl.ds(col, WIN)],
                bufs[q % 2], sems[q % 2])

        nq = 128 // SC_CHUNK
        handles = {0: start(0)}
        for q in range(nq):
            if q + 1 < nq:
                handles[q + 1] = start(q + 1)
            handles[q].wait()
            buf = bufs[q % 2]
            for r in range(SC_CHUNK):
                labr = plsc.load_gather(
                    idx_v, [z16, jnp.full((16,), q * SC_CHUNK + r, jnp.int32)])
                for k in range(WIN // 16):
                    v = buf[r, pl.ds(k * 16, 16)]
                    plsc.addupdate_scatter(acc, [labr, iota16 + (k * 16)], v)
        return carry

    lax.fori_loop(0, LROWS, lr_body, 0)
    pltpu.sync_copy(acc, out_hbm.at[pl.ds(c * NUM_LEAVES, NUM_LEAVES),
                                    pl.ds(col, WIN)])


@functools.lru_cache(maxsize=1)
def _sc_scatter():
    return pl.kernel(
        _sc_scatter_body,
        out_type=jax.ShapeDtypeStruct((SC_CORES * NUM_LEAVES, D), jnp.float32),
        mesh=plsc.VectorSubcoreMesh(core_axis_name="c", subcore_axis_name="s"),
        compiler_params=pltpu.CompilerParams(needs_layout_passes=False),
        scratch_types=[
            pltpu.VMEM((1, 128), jnp.int32),
            pltpu.VMEM((SC_CHUNK, WIN), jnp.float32),
            pltpu.VMEM((SC_CHUNK, WIN), jnp.float32),
            pltpu.SemaphoreType.DMA,
            pltpu.SemaphoreType.DMA,
            pltpu.VMEM((NUM_LEAVES, WIN), jnp.float32),
        ],
    )


def _vmf_weights(ave, amount, sums, counts):
    new_amount = amount + counts
    new_ave = (ave * amount + sums) / new_amount
    r2 = jnp.sum(new_ave * new_ave, axis=1, keepdims=True)
    r = jnp.sqrt(r2)
    r_c = jnp.clip(r, 1e-6, 1.0 - 1e-6)
    kappa = r_c * (D - r_c * r_c) / (1.0 - r_c * r_c)
    scale = kappa / jnp.maximum(r, 1e-12) / TEMPERATURE
    w = new_ave * scale
    pt = jnp.sum(jnp.sum(sums * new_ave, axis=1, keepdims=True) * scale)
    return w, pt


def _prep_body(lab_ref, p_ref, ave_root_ref, ave_super_ref, ave_leaf_ref,
               amt_root_ref, amt_super_ref, amt_leaf_ref, w_ref, pt_ref):
    lab = lab_ref[...]
    node_iota = lax.broadcasted_iota(jnp.int32, (1024, B), 0)
    onehot = (node_iota == lab).astype(jnp.float32)
    counts_all = jnp.sum(onehot, axis=1, keepdims=True)
    counts_leaf = counts_all[:NUM_LEAVES]

    leaf_sums = p_ref[0] + p_ref[1]

    io_s = lax.broadcasted_iota(jnp.int32, (NUM_SUPER, NUM_LEAVES), 0)
    io_l = lax.broadcasted_iota(jnp.int32, (NUM_SUPER, NUM_LEAVES), 1)
    sel = ((io_l >= 10 * io_s) & (io_l < 10 * io_s + 10)).astype(jnp.float32)
    super_sums = jnp.dot(sel, leaf_sums, preferred_element_type=jnp.float32)
    super_counts = jnp.dot(sel, counts_leaf, preferred_element_type=jnp.float32)
    root_sum = jnp.sum(super_sums, axis=0, keepdims=True)
    root_count = jnp.sum(counts_leaf, axis=0, keepdims=True)

    w_root, pt0 = _vmf_weights(ave_root_ref[...], amt_root_ref[...], root_sum, root_count)
    w_super, pt1 = _vmf_weights(ave_super_ref[...], amt_super_ref[...], super_sums, super_counts)
    w_leaf, pt2 = _vmf_weights(ave_leaf_ref[...], amt_leaf_ref[...], leaf_sums, counts_leaf)

    w_ref[...] = jnp.concatenate(
        [w_root, w_super, w_leaf,
         jnp.zeros((N_PAD - NUM_NODES, D), jnp.float32)], axis=0)
    pt_ref[0, 0] = pt0 + pt1 + pt2


def _loss_body(feat_ref, w_ref, pt_ref, out_ref, acc_ref):
    i = pl.program_id(0)

    @pl.when(i == 0)
    def _():
        acc_ref[0, 0] = 0.0

    z = lax.dot_general(
        feat_ref[...], w_ref[...],
        dimension_numbers=(((1,), (1,)), ((), ())),
        preferred_element_type=jnp.float32)
    sp = jnp.maximum(z, 0.0) + jnp.log1p(jnp.exp(-jnp.abs(z)))
    mask = lax.broadcasted_iota(jnp.int32, (BM, N_PAD), 1) < NUM_NODES
    acc_ref[0, 0] += jnp.sum(jnp.where(mask, sp, 0.0))

    @pl.when(i == pl.num_programs(0) - 1)
    def _():
        val = (acc_ref[0, 0] - pt_ref[0, 0]) / float(B * NUM_NODES)
        out_ref[...] = jnp.full((1, 1), val, jnp.float32)


def _run_prep(labels, partials, ave, amount):
    lab2 = labels.reshape(1, B).astype(jnp.int32)
    amount_col = amount.reshape(NUM_NODES, 1)
    return pl.pallas_call(
        _prep_body,
        out_shape=(
            jax.ShapeDtypeStruct((N_PAD, D), jnp.float32),
            jax.ShapeDtypeStruct((1, 1), jnp.float32),
        ),
        out_specs=(
            pl.BlockSpec(memory_space=pltpu.VMEM),
            pl.BlockSpec(memory_space=pltpu.SMEM),
        ),
    )(lab2, partials.reshape(SC_CORES, NUM_LEAVES, D),
      ave[0:1], ave[1:1 + NUM_SUPER], ave[1 + NUM_SUPER:NUM_NODES],
      amount_col[0:1], amount_col[1:1 + NUM_SUPER],
      amount_col[1 + NUM_SUPER:NUM_NODES])


def _run_loss(features, w, pt):
    grid = (B // BM,)
    return pl.pallas_call(
        _loss_body,
        grid=grid,
        in_specs=[
            pl.BlockSpec((BM, D), lambda i: (i, 0)),
            pl.BlockSpec((N_PAD, D), lambda i: (0, 0)),
            pl.BlockSpec(memory_space=pltpu.SMEM),
        ],
        out_specs=pl.BlockSpec((1, 1), lambda i: (0, 0)),
        out_shape=jax.ShapeDtypeStruct((1, 1), jnp.float32),
        scratch_shapes=[pltpu.SMEM((1, 1), jnp.float32)],
    )(features, w, pt)


def kernel(features, leaf_labels, Ave, Amount):
    zeros_block = jnp.zeros((NUM_LEAVES, WIN), jnp.float32)
    labs2 = leaf_labels.reshape(B // 128, 128).astype(jnp.int32)
    partials = _sc_scatter()(features, labs2, zeros_block)
    w, pt = _run_prep(leaf_labels, partials, Ave, Amount)
    out = _run_loss(features, w, pt)
    return out.reshape(())


# speedup vs baseline: 11.6846x; 1.0708x over previous
import functools

import jax
import jax.numpy as jnp
from jax import lax
from jax.experimental import pallas as pl
from jax.experimental.pallas import tpu as pltpu
from jax.experimental.pallas import tpu_sc as plsc

B = 4096
D = 2048
NUM_LEAVES = 1000
NUM_SUPER = 100
NUM_NODES = 1 + NUM_SUPER + NUM_LEAVES
N_PAD = 1152
TEMPERATURE = 1.0

SC_CORES = 2
SC_SUBCORES = 16
WIN = D // SC_SUBCORES
ROWS_PER_CORE = B // SC_CORES
SC_CHUNK = 8
LROWS = ROWS_PER_CORE // 128

BM = 512


def _sc_scatter_body(feat_hbm, lab_hbm, zeros_hbm, out_hbm,
                     idx_v, rows_a, rows_b, sem_a, sem_b, acc):
    c = lax.axis_index("c")
    s = lax.axis_index("s")
    col = s * WIN
    pltpu.sync_copy(zeros_hbm, acc)
    iota16 = lax.iota(jnp.int32, 16)
    z16 = jnp.zeros((16,), jnp.int32)
    bufs = (rows_a, rows_b)
    sems = (sem_a, sem_b)

    def lr_body(lr, carry):
        pltpu.sync_copy(lab_hbm.at[pl.ds(c * LROWS + lr, 1)], idx_v)
        row0 = c * ROWS_PER_CORE + lr * 128

        def start(q):
            return pltpu.async_copy(
                feat_hbm.at[pl.ds(row0 + q * SC_CHUNK, SC_CHUNK),
                            pl.ds(col, WIN)],
                bufs[q % 2], sems[q % 2])

        nq = 128 // SC_CHUNK
        handles = {0: start(0)}
        for q in range(nq):
            if q + 1 < nq:
                handles[q + 1] = start(q + 1)
            handles[q].wait()
            buf = bufs[q % 2]

            @plsc.parallel_loop(0, SC_CHUNK, unroll=SC_CHUNK)
            def _(r, _q=q, _buf=buf):
                labr = plsc.load_gather(
                    idx_v, [z16, jnp.full((16,), _q * SC_CHUNK + r, jnp.int32)])
                for k in range(WIN // 16):
                    v = _buf[r, pl.ds(k * 16, 16)]
                    plsc.addupdate_scatter(acc, [labr, iota16 + (k * 16)], v)
        return carry

    lax.fori_loop(0, LROWS, lr_body, 0)
    pltpu.sync_copy(acc, out_hbm.at[pl.ds(c * NUM_LEAVES, NUM_LEAVES),
                                    pl.ds(col, WIN)])


@functools.lru_cache(maxsize=1)
def _sc_scatter():
    return pl.kernel(
        _sc_scatter_body,
        out_type=jax.ShapeDtypeStruct((SC_CORES * NUM_LEAVES, D), jnp.float32),
        mesh=plsc.VectorSubcoreMesh(core_axis_name="c", subcore_axis_name="s"),
        compiler_params=pltpu.CompilerParams(needs_layout_passes=False),
        scratch_types=[
            pltpu.VMEM((1, 128), jnp.int32),
            pltpu.VMEM((SC_CHUNK, WIN), jnp.float32),
            pltpu.VMEM((SC_CHUNK, WIN), jnp.float32),
            pltpu.SemaphoreType.DMA,
            pltpu.SemaphoreType.DMA,
            pltpu.VMEM((NUM_LEAVES, WIN), jnp.float32),
        ],
    )


def _vmf_weights(ave, amount, sums, counts):
    new_amount = amount + counts
    new_ave = (ave * amount + sums) / new_amount
    r2 = jnp.sum(new_ave * new_ave, axis=1, keepdims=True)
    r = jnp.sqrt(r2)
    r_c = jnp.clip(r, 1e-6, 1.0 - 1e-6)
    kappa = r_c * (D - r_c * r_c) / (1.0 - r_c * r_c)
    scale = kappa / jnp.maximum(r, 1e-12) / TEMPERATURE
    w = new_ave * scale
    pt = jnp.sum(jnp.sum(sums * new_ave, axis=1, keepdims=True) * scale)
    return w, pt


def _prep_body(lab_ref, p_ref, ave_root_ref, ave_super_ref, ave_leaf_ref,
               amt_root_ref, amt_super_ref, amt_leaf_ref, w_ref, pt_ref):
    lab = lab_ref[...]
    node_iota = lax.broadcasted_iota(jnp.int32, (1024, B), 0)
    onehot = (node_iota == lab).astype(jnp.float32)
    counts_all = jnp.sum(onehot, axis=1, keepdims=True)
    counts_leaf = counts_all[:NUM_LEAVES]

    leaf_sums = p_ref[0] + p_ref[1]

    io_s = lax.broadcasted_iota(jnp.int32, (NUM_SUPER, NUM_LEAVES), 0)
    io_l = lax.broadcasted_iota(jnp.int32, (NUM_SUPER, NUM_LEAVES), 1)
    sel = ((io_l >= 10 * io_s) & (io_l < 10 * io_s + 10)).astype(jnp.float32)
    super_sums = jnp.dot(sel, leaf_sums, preferred_element_type=jnp.float32)
    super_counts = jnp.dot(sel, counts_leaf, preferred_element_type=jnp.float32)
    root_sum = jnp.sum(super_sums, axis=0, keepdims=True)
    root_count = jnp.sum(counts_leaf, axis=0, keepdims=True)

    w_root, pt0 = _vmf_weights(ave_root_ref[...], amt_root_ref[...], root_sum, root_count)
    w_super, pt1 = _vmf_weights(ave_super_ref[...], amt_super_ref[...], super_sums, super_counts)
    w_leaf, pt2 = _vmf_weights(ave_leaf_ref[...], amt_leaf_ref[...], leaf_sums, counts_leaf)

    w_ref[...] = jnp.concatenate(
        [w_root, w_super, w_leaf,
         jnp.zeros((N_PAD - NUM_NODES, D), jnp.float32)], axis=0)
    pt_ref[0, 0] = pt0 + pt1 + pt2


def _loss_body(feat_ref, w_ref, pt_ref, out_ref, acc_ref):
    i = pl.program_id(0)

    @pl.when(i == 0)
    def _():
        acc_ref[0, 0] = 0.0

    z = lax.dot_general(
        feat_ref[...], w_ref[...],
        dimension_numbers=(((1,), (1,)), ((), ())),
        preferred_element_type=jnp.float32)
    sp = jnp.maximum(z, 0.0) + jnp.log1p(jnp.exp(-jnp.abs(z)))
    mask = lax.broadcasted_iota(jnp.int32, (BM, N_PAD), 1) < NUM_NODES
    acc_ref[0, 0] += jnp.sum(jnp.where(mask, sp, 0.0))

    @pl.when(i == pl.num_programs(0) - 1)
    def _():
        val = (acc_ref[0, 0] - pt_ref[0, 0]) / float(B * NUM_NODES)
        out_ref[...] = jnp.full((1, 1), val, jnp.float32)


def _run_prep(labels, partials, ave, amount):
    lab2 = labels.reshape(1, B).astype(jnp.int32)
    amount_col = amount.reshape(NUM_NODES, 1)
    return pl.pallas_call(
        _prep_body,
        out_shape=(
            jax.ShapeDtypeStruct((N_PAD, D), jnp.float32),
            jax.ShapeDtypeStruct((1, 1), jnp.float32),
        ),
        out_specs=(
            pl.BlockSpec(memory_space=pltpu.VMEM),
            pl.BlockSpec(memory_space=pltpu.SMEM),
        ),
    )(lab2, partials.reshape(SC_CORES, NUM_LEAVES, D),
      ave[0:1], ave[1:1 + NUM_SUPER], ave[1 + NUM_SUPER:NUM_NODES],
      amount_col[0:1], amount_col[1:1 + NUM_SUPER],
      amount_col[1 + NUM_SUPER:NUM_NODES])


def _run_loss(features, w, pt):
    grid = (B // BM,)
    return pl.pallas_call(
        _loss_body,
        grid=grid,
        in_specs=[
            pl.BlockSpec((BM, D), lambda i: (i, 0)),
            pl.BlockSpec((N_PAD, D), lambda i: (0, 0)),
            pl.BlockSpec(memory_space=pltpu.SMEM),
        ],
        out_specs=pl.BlockSpec((1, 1), lambda i: (0, 0)),
        out_shape=jax.ShapeDtypeStruct((1, 1), jnp.float32),
        scratch_shapes=[pltpu.SMEM((1, 1), jnp.float32)],
    )(features, w, pt)


def kernel(features, leaf_labels, Ave, Amount):
    zeros_block = jnp.zeros((NUM_LEAVES, WIN), jnp.float32)
    labs2 = leaf_labels.reshape(B // 128, 128).astype(jnp.int32)
    partials = _sc_scatter()(features, labs2, zeros_block)
    w, pt = _run_prep(leaf_labels, partials, Ave, Amount)
    out = _run_loss(features, w, pt)
    return out.reshape(())
